# baseline (speedup 1.0000x reference)
import jax
import jax.numpy as jnp
from jax import lax
from jax.experimental import pallas as pl
from jax.experimental.pallas import tpu as pltpu
from jax.experimental.pallas import tpu_sc as plsc

_N = 10000
_E = 320000
_D = 128
_M = 10112
_K = 128
_CH = 80
_HC = 40
_DUMMY = _M - 1
_RPT = _M // 16
_BM = 2528

_MESH = dict(core_axis_name="c", subcore_axis_name="s")


def _seg_pass():
    scratch = [
        pltpu.VMEM((_HC, 2, _K), jnp.int32),
        pltpu.VMEM((_K, _D), jnp.float32),
        pltpu.VMEM((_K, _D), jnp.float32),
        pltpu.VMEM_SHARED((_M, _D), jnp.float32),
        pltpu.SemaphoreType.DMA,
        pltpu.SemaphoreType.DMA,
        pltpu.SemaphoreType.DMA,
        pltpu.SemaphoreType.DMA,
    ]

    def body(tbl, gs, z128, psum, idv, rows0, rows1, acc,
             semg0, semg1, sems0, sems1):
        cid = lax.axis_index("c")
        sid = lax.axis_index("s")
        wid = cid * 16 + sid
        r0 = sid * _RPT
        rowsl = (rows0, rows1)
        semgl = (semg0, semg1)
        semsl = (sems0, sems1)
        pltpu.sync_copy(z128.at[pl.ds(r0, _RPT)], acc.at[pl.ds(r0, _RPT)])
        plsc.subcore_barrier()

        def step(j, carry):
            for r in range(2):
                def branch(r=r):
                    rows_c, semg_c, sems_c = rowsl[r], semgl[r], semsl[r]
                    rows_n, semg_n, sems_n = (
                        rowsl[1 - r], semgl[1 - r], semsl[1 - r])
                    nxt = j + 1

                    @pl.when(nxt < _HC)
                    def _():
                        @pl.when(j >= 1)
                        def _():
                            pltpu.make_async_copy(
                                rows_n, acc.at[idv.at[0, 1]], sems_n).wait()
                        pltpu.async_copy(
                            tbl.at[idv.at[nxt, 0]], rows_n, semg_n)

                    pltpu.make_async_copy(
                        tbl.at[idv.at[j, 0]], rows_c, semg_c).wait()
                    pltpu.async_copy(
                        rows_c, acc.at[idv.at[j, 1]], sems_c, add=True)
                pl.when(lax.rem(j, 2) == r)(branch)
            return carry

        for h in range(_CH // _HC):
            pltpu.sync_copy(gs.at[wid, pl.ds(h * _HC, _HC)], idv)
            pltpu.async_copy(tbl.at[idv.at[0, 0]], rows0, semg0)
            lax.fori_loop(0, _HC, step, 0)
            pltpu.make_async_copy(rows0, acc.at[idv.at[0, 1]], sems0).wait()
            pltpu.make_async_copy(rows1, acc.at[idv.at[0, 1]], sems1).wait()
        plsc.subcore_barrier()
        o0 = cid * _M + r0
        pltpu.sync_copy(acc.at[pl.ds(r0, _RPT)], psum.at[pl.ds(o0, _RPT)])

    return pl.kernel(
        body, mesh=plsc.VectorSubcoreMesh(**_MESH),
        out_type=jax.ShapeDtypeStruct((2 * _M, _D), jnp.float32),
        scratch_types=scratch)


def _cnt_pass():
    scratch = [
        pltpu.VMEM((_HC, 2, _K), jnp.int32),
        pltpu.VMEM((_K, _D), jnp.float32),
        pltpu.VMEM_SHARED((_M, _D), jnp.float32),
        pltpu.SemaphoreType.DMA,
        pltpu.SemaphoreType.DMA,
    ]
    out_type = [jax.ShapeDtypeStruct((2 * _M, _D), jnp.float32),
                jax.ShapeDtypeStruct((2 * _M, _D), jnp.float32)]

    def body(gs, z128, ones, cn_out, ce_out, idv, ones_v, acc,
             sems0, sems1):
        cid = lax.axis_index("c")
        sid = lax.axis_index("s")
        wid = cid * 16 + sid
        r0 = sid * _RPT
        o0 = cid * _M + r0
        semsl = (sems0, sems1)
        pltpu.sync_copy(ones, ones_v)
        for row, out in ((0, cn_out), (1, ce_out)):
            pltpu.sync_copy(z128.at[pl.ds(r0, _RPT)], acc.at[pl.ds(r0, _RPT)])
            plsc.subcore_barrier()

            def step(j, carry, row=row):
                for r in range(2):
                    def branch(r=r, row=row):
                        sems_c = semsl[r]

                        @pl.when(j >= 2)
                        def _():
                            pltpu.make_async_copy(
                                ones_v, acc.at[idv.at[0, row]],
                                sems_c).wait()
                        pltpu.async_copy(
                            ones_v, acc.at[idv.at[j, row]], sems_c,
                            add=True)
                    pl.when(lax.rem(j, 2) == r)(branch)
                return carry

            for h in range(_CH // _HC):
                pltpu.sync_copy(gs.at[wid, pl.ds(h * _HC, _HC)], idv)
                lax.fori_loop(0, _HC, step, 0)
                pltpu.make_async_copy(ones_v, acc.at[idv.at[0, row]],
                                      sems0).wait()
                pltpu.make_async_copy(ones_v, acc.at[idv.at[0, row]],
                                      sems1).wait()
            plsc.subcore_barrier()
            pltpu.sync_copy(acc.at[pl.ds(r0, _RPT)], out.at[pl.ds(o0, _RPT)])

    return pl.kernel(
        body, mesh=plsc.VectorSubcoreMesh(**_MESH),
        out_type=out_type, scratch_types=scratch)


def _mean_mm(relu):
    def body(pa, pb, ca, cb, w, b, out):
        s = pa[...] + pb[...]
        cnt = ca[..., 0:1] + cb[..., 0:1]
        inv = 1.0 / jnp.maximum(cnt, 1.0)
        h = jnp.dot(s * inv, w[...], preferred_element_type=jnp.float32)
        h = h + b[...]
        out[...] = jnp.maximum(h, 0.0) if relu else h

    nb = _M // _BM
    return pl.pallas_call(
        body,
        grid=(nb,),
        in_specs=[
            pl.BlockSpec((_BM, _D), lambda i: (i, 0)),
            pl.BlockSpec((_BM, _D), lambda i, nb=nb: (i + nb, 0)),
            pl.BlockSpec((_BM, _D), lambda i: (i, 0)),
            pl.BlockSpec((_BM, _D), lambda i, nb=nb: (i + nb, 0)),
            pl.BlockSpec((_D, _D), lambda i: (0, 0)),
            pl.BlockSpec((1, _D), lambda i: (0, 0)),
        ],
        out_specs=pl.BlockSpec((_BM, _D), lambda i: (i, 0)),
        out_shape=jax.ShapeDtypeStruct((_M, _D), jnp.float32),
    )


def kernel(x, ei, W1_e, b1_e, W1_n, b1_n, W2_e, b2_e, W2_n, b2_n):
    ids = ei.astype(jnp.int32).reshape(2, 32, _E // 32)
    ids = jnp.pad(ids, ((0, 0), (0, 0), (0, _CH * _K - _E // 32)),
                  constant_values=_DUMMY)
    ids = ids.reshape(2, 32, _CH, _K)
    gs_ne = jnp.stack([ids[0], ids[1]], axis=2)
    gs_en = jnp.stack([ids[1], ids[0]], axis=2)
    x_pad = jnp.pad(x, ((0, _M - _N), (0, 0)))
    z128 = jnp.zeros((_M, _D), jnp.float32)
    ones = jnp.ones((_K, _D), jnp.float32)
    b1_e2 = b1_e.reshape(1, _D)
    b1_n2 = b1_n.reshape(1, _D)
    b2_e2 = b2_e.reshape(1, _D)
    b2_n2 = b2_n.reshape(1, _D)

    seg = _seg_pass()
    mm_e = _mean_mm(False)
    mm_n = _mean_mm(True)

    cn, ce = _cnt_pass()(gs_ne, z128, ones)
    esum1 = seg(x_pad, gs_ne, z128)
    efeat1 = mm_e(esum1, esum1, ce, ce, W1_e, b1_e2)
    nsum1 = seg(efeat1, gs_en, z128)
    z1 = mm_n(nsum1, nsum1, cn, cn, W1_n, b1_n2)
    esum2 = seg(z1, gs_ne, z128)
    efeat2 = mm_e(esum2, esum2, ce, ce, W2_e, b2_e2)
    nsum2 = seg(efeat2, gs_en, z128)
    z2 = mm_n(nsum2, nsum2, cn, cn, W2_n, b2_n2)
    return z2[:_N]


# speedup vs baseline: 1.5405x; 1.5405x over previous
import jax
import jax.numpy as jnp
from jax import lax
from jax.experimental import pallas as pl
from jax.experimental.pallas import tpu as pltpu
from jax.experimental.pallas import tpu_sc as plsc

_N = 10000
_E = 320000
_D = 128
_M = 10112
_K = 128
_CH = 79
_DUMMY = _M - 1
_RPT = _M // 16
_BM = 2528

_MESH = dict(core_axis_name="c", subcore_axis_name="s")


def _seg_pass():
    scratch = [
        pltpu.VMEM((2, _K), jnp.int32),
        pltpu.VMEM((2, _K), jnp.int32),
        pltpu.VMEM((2, _K), jnp.int32),
        pltpu.VMEM((2, _K), jnp.int32),
        pltpu.VMEM((_K, _D), jnp.float32),
        pltpu.VMEM((_K, _D), jnp.float32),
        pltpu.VMEM_SHARED((_M, _D), jnp.float32),
        pltpu.SemaphoreType.DMA,
        pltpu.SemaphoreType.DMA,
        pltpu.SemaphoreType.DMA,
        pltpu.SemaphoreType.DMA,
        pltpu.SemaphoreType.DMA,
        pltpu.SemaphoreType.DMA,
        pltpu.SemaphoreType.DMA,
        pltpu.SemaphoreType.DMA,
    ]

    def body(tbl, gs, z128, psum, gsb0, gsb1, gsb2, gsb3, rows0, rows1, acc,
             semg0, semg1, sems0, sems1, semi0, semi1, semi2, semi3):
        cid = lax.axis_index("c")
        sid = lax.axis_index("s")
        wid = cid * 16 + sid
        r0 = sid * _RPT
        gsl = (gsb0, gsb1, gsb2, gsb3)
        semil = (semi0, semi1, semi2, semi3)
        rowsl = (rows0, rows1)
        semgl = (semg0, semg1)
        semsl = (sems0, sems1)
        pltpu.sync_copy(z128.at[pl.ds(r0, _RPT)], acc.at[pl.ds(r0, _RPT)])
        pltpu.sync_copy(gs.at[wid, 0], gsb0)
        pltpu.sync_copy(gs.at[wid, 1], gsb1)
        pltpu.sync_copy(gs.at[wid, 2], gsb2)
        pltpu.async_copy(tbl.at[gsb0.at[0]], rows0, semg0)
        plsc.subcore_barrier()

        def step(j, carry):
            m4 = lax.rem(j, 4)
            for r in range(4):
                def branch(r=r):
                    cur = gsl[r]
                    nxtg = gsl[(r + 1) % 4]
                    ldb = gsl[(r + 3) % 4]
                    rows_c, semg_c, sems_c = (
                        rowsl[r % 2], semgl[r % 2], semsl[r % 2])
                    rows_n, semg_n, sems_n = (
                        rowsl[(r + 1) % 2], semgl[(r + 1) % 2],
                        semsl[(r + 1) % 2])
                    nxt = j + 1

                    @pl.when(nxt < _CH)
                    def _():
                        @pl.when(j >= 1)
                        def _():
                            pltpu.make_async_copy(
                                rows_n, acc.at[nxtg.at[1]], sems_n).wait()
                        @pl.when(j >= 2)
                        def _():
                            pltpu.make_async_copy(
                                gs.at[wid, nxt], nxtg,
                                semil[(r + 1) % 4]).wait()
                        pltpu.async_copy(tbl.at[nxtg.at[0]], rows_n, semg_n)

                    @pl.when(j + 3 < _CH)
                    def _():
                        pltpu.async_copy(gs.at[wid, j + 3], ldb,
                                         semil[(r + 3) % 4])

                    pltpu.make_async_copy(
                        tbl.at[cur.at[0]], rows_c, semg_c).wait()
                    pltpu.async_copy(rows_c, acc.at[cur.at[1]], sems_c,
                                     add=True)
                pl.when(m4 == r)(branch)
            return carry

        lax.fori_loop(0, _CH, step, 0)
        pltpu.make_async_copy(rows1, acc.at[gsb1.at[1]], sems1).wait()
        pltpu.make_async_copy(rows0, acc.at[gsb0.at[1]], sems0).wait()
        plsc.subcore_barrier()
        o0 = cid * _M + r0
        pltpu.sync_copy(acc.at[pl.ds(r0, _RPT)], psum.at[pl.ds(o0, _RPT)])

    return pl.kernel(
        body, mesh=plsc.VectorSubcoreMesh(**_MESH),
        out_type=jax.ShapeDtypeStruct((2 * _M, _D), jnp.float32),
        scratch_types=scratch)


def _cnt_pass():
    scratch = [
        pltpu.VMEM((2, _K), jnp.int32),
        pltpu.VMEM((2, _K), jnp.int32),
        pltpu.VMEM((2, _K), jnp.int32),
        pltpu.VMEM((2, _K), jnp.int32),
        pltpu.VMEM((_K, _D), jnp.float32),
        pltpu.VMEM_SHARED((_M, _D), jnp.float32),
        pltpu.SemaphoreType.DMA,
        pltpu.SemaphoreType.DMA,
        pltpu.SemaphoreType.DMA,
        pltpu.SemaphoreType.DMA,
        pltpu.SemaphoreType.DMA,
        pltpu.SemaphoreType.DMA,
    ]
    out_type = [jax.ShapeDtypeStruct((2 * _M, _D), jnp.float32),
                jax.ShapeDtypeStruct((2 * _M, _D), jnp.float32)]

    def body(gs, z128, ones, cn_out, ce_out, iv0, iv1, iv2, iv3, ones_v,
             acc, sems0, sems1, semi0, semi1, semi2, semi3):
        cid = lax.axis_index("c")
        sid = lax.axis_index("s")
        wid = cid * 16 + sid
        r0 = sid * _RPT
        o0 = cid * _M + r0
        ivl = (iv0, iv1, iv2, iv3)
        semil = (semi0, semi1, semi2, semi3)
        semsl = (sems0, sems1)
        pltpu.sync_copy(ones, ones_v)
        for row, out in ((0, cn_out), (1, ce_out)):
            pltpu.sync_copy(z128.at[pl.ds(r0, _RPT)], acc.at[pl.ds(r0, _RPT)])
            pltpu.sync_copy(gs.at[wid, 0], iv0)
            pltpu.sync_copy(gs.at[wid, 1], iv1)
            plsc.subcore_barrier()

            def step(j, carry, row=row):
                m4 = lax.rem(j, 4)
                for r in range(4):
                    def branch(r=r, row=row):
                        cur = ivl[r]
                        ldb = ivl[(r + 2) % 4]
                        sems_c = semsl[r % 2]

                        @pl.when(j >= 2)
                        def _():
                            pltpu.make_async_copy(
                                ones_v, acc.at[ldb.at[row]], sems_c).wait()

                        @pl.when(j + 2 < _CH)
                        def _():
                            pltpu.async_copy(gs.at[wid, j + 2], ldb,
                                             semil[(r + 2) % 4])

                        @pl.when(j >= 2)
                        def _():
                            pltpu.make_async_copy(
                                gs.at[wid, j], cur, semil[r]).wait()

                        pltpu.async_copy(
                            ones_v, acc.at[cur.at[row]], sems_c, add=True)
                    pl.when(m4 == r)(branch)
                return carry

            lax.fori_loop(0, _CH, step, 0)
            pltpu.make_async_copy(ones_v, acc.at[iv0.at[row]], sems0).wait()
            pltpu.make_async_copy(ones_v, acc.at[iv1.at[row]], sems1).wait()
            plsc.subcore_barrier()
            pltpu.sync_copy(acc.at[pl.ds(r0, _RPT)], out.at[pl.ds(o0, _RPT)])

    return pl.kernel(
        body, mesh=plsc.VectorSubcoreMesh(**_MESH),
        out_type=out_type, scratch_types=scratch)


def _mean_mm(relu):
    def body(pa, pb, ca, cb, w, b, out):
        s = pa[...] + pb[...]
        cnt = ca[..., 0:1] + cb[..., 0:1]
        inv = 1.0 / jnp.maximum(cnt, 1.0)
        h = jnp.dot(s * inv, w[...], preferred_element_type=jnp.float32)
        h = h + b[...]
        out[...] = jnp.maximum(h, 0.0) if relu else h

    nb = _M // _BM
    return pl.pallas_call(
        body,
        grid=(nb,),
        in_specs=[
            pl.BlockSpec((_BM, _D), lambda i: (i, 0)),
            pl.BlockSpec((_BM, _D), lambda i, nb=nb: (i + nb, 0)),
            pl.BlockSpec((_BM, _D), lambda i: (i, 0)),
            pl.BlockSpec((_BM, _D), lambda i, nb=nb: (i + nb, 0)),
            pl.BlockSpec((_D, _D), lambda i: (0, 0)),
            pl.BlockSpec((1, _D), lambda i: (0, 0)),
        ],
        out_specs=pl.BlockSpec((_BM, _D), lambda i: (i, 0)),
        out_shape=jax.ShapeDtypeStruct((_M, _D), jnp.float32),
    )


def kernel(x, ei, W1_e, b1_e, W1_n, b1_n, W2_e, b2_e, W2_n, b2_n):
    ids = ei.astype(jnp.int32).reshape(2, 32, _E // 32)
    ids = jnp.pad(ids, ((0, 0), (0, 0), (0, _CH * _K - _E // 32)),
                  constant_values=_DUMMY)
    ids = ids.reshape(2, 32, _CH, _K)
    gs_ne = jnp.stack([ids[0], ids[1]], axis=2)
    gs_en = jnp.stack([ids[1], ids[0]], axis=2)
    x_pad = jnp.pad(x, ((0, _M - _N), (0, 0)))
    z128 = jnp.zeros((_M, _D), jnp.float32)
    ones = jnp.ones((_K, _D), jnp.float32)
    b1_e2 = b1_e.reshape(1, _D)
    b1_n2 = b1_n.reshape(1, _D)
    b2_e2 = b2_e.reshape(1, _D)
    b2_n2 = b2_n.reshape(1, _D)

    seg = _seg_pass()
    mm_e = _mean_mm(False)
    mm_n = _mean_mm(True)

    cn, ce = _cnt_pass()(gs_ne, z128, ones)
    esum1 = seg(x_pad, gs_ne, z128)
    efeat1 = mm_e(esum1, esum1, ce, ce, W1_e, b1_e2)
    nsum1 = seg(efeat1, gs_en, z128)
    z1 = mm_n(nsum1, nsum1, cn, cn, W1_n, b1_n2)
    esum2 = seg(z1, gs_ne, z128)
    efeat2 = mm_e(esum2, esum2, ce, ce, W2_e, b2_e2)
    nsum2 = seg(efeat2, gs_en, z128)
    z2 = mm_n(nsum2, nsum2, cn, cn, W2_n, b2_n2)
    return z2[:_N]
